# baseline (speedup 1.0000x reference)
import functools

import jax
import jax.numpy as jnp
from jax import lax
from jax.experimental import pallas as pl
from jax.experimental.pallas import tpu as pltpu
from jax.experimental.pallas import tpu_sc as plsc

HID = 4096
NE = 64
TOPK = 8
NTOK = 8192
BLK = 512
NC, NS, L = 2, 16, 16
NW = NC * NS
C = 4
CH = NTOK // C
TPW = CH // NW
NEG = -3.0e38


def _matmul_kernel(x_ref, w_ref, out_ref):
    logits = lax.dot_general(
        w_ref[...], x_ref[...], (((1,), (1,)), ((), ())),
        preferred_element_type=jnp.float32)
    for c in range(BLK // TPW):
        out_ref[c] = logits[:, c * TPW:(c + 1) * TPW]


_sc_mesh = plsc.VectorSubcoreMesh(core_axis_name="c", subcore_axis_name="s")


@functools.partial(
    pl.kernel,
    mesh=_sc_mesh,
    out_type=[
        jax.ShapeDtypeStruct((NW, TOPK, TPW), jnp.float32),
        jax.ShapeDtypeStruct((NW, TOPK, TPW), jnp.int32),
    ],
    scratch_types=[
        pltpu.VMEM((NE, TPW), jnp.float32),
        pltpu.VMEM((TOPK, TPW), jnp.float32),
        pltpu.VMEM((TOPK, TPW), jnp.int32),
    ],
)
def _topk_kernel(lg_hbm, wout_hbm, iout_hbm, lg_v, wv, iv):
    wid = lax.axis_index("s") * NC + lax.axis_index("c")
    pltpu.sync_copy(lg_hbm.at[wid], lg_v)

    def group(g, carry):
        off = g * L
        vals = [jnp.full((L,), NEG, jnp.float32) for _ in range(TOPK)]
        idxs = [jnp.zeros((L,), jnp.int32) for _ in range(TOPK)]
        for e in range(NE):
            v = lg_v[e, pl.ds(off, L)]
            ev = jnp.full((L,), e, jnp.int32)
            c = v > vals[TOPK - 1]
            vals[TOPK - 1] = jnp.where(c, v, vals[TOPK - 1])
            idxs[TOPK - 1] = jnp.where(c, ev, idxs[TOPK - 1])
            for j in range(TOPK - 1, 0, -1):
                c2 = vals[j] > vals[j - 1]
                vhi = jnp.maximum(vals[j - 1], vals[j])
                vlo = jnp.minimum(vals[j - 1], vals[j])
                ihi = jnp.where(c2, idxs[j], idxs[j - 1])
                ilo = jnp.where(c2, idxs[j - 1], idxs[j])
                vals[j - 1], vals[j] = vhi, vlo
                idxs[j - 1], idxs[j] = ihi, ilo
        es = [jnp.exp(v - vals[0]) for v in vals]
        s = es[0]
        for j in range(1, TOPK):
            s = s + es[j]
        for j in range(TOPK):
            wv[j, pl.ds(off, L)] = es[j] / s
            iv[j, pl.ds(off, L)] = idxs[j]
        return carry

    lax.fori_loop(0, TPW // L, group, 0)
    pltpu.sync_copy(wv, wout_hbm.at[wid])
    pltpu.sync_copy(iv, iout_hbm.at[wid])


_matmul_call = pl.pallas_call(
    _matmul_kernel,
    grid=(CH // BLK,),
    in_specs=[
        pl.BlockSpec((BLK, HID), lambda t: (t, 0)),
        pl.BlockSpec((NE, HID), lambda t: (0, 0)),
    ],
    out_specs=pl.BlockSpec((BLK // TPW, NE, TPW), lambda t: (t, 0, 0)),
    out_shape=jax.ShapeDtypeStruct((CH // TPW, NE, TPW), jnp.float32),
    compiler_params=pltpu.CompilerParams(
        dimension_semantics=("arbitrary",)),
)


@jax.jit
def kernel(hidden_states, weight):
    b, s, h = hidden_states.shape
    n = b * s
    x = hidden_states.reshape(n, h).astype(jnp.float32)
    w = weight.astype(jnp.float32)
    ws, inds = [], []
    for c in range(C):
        lg = _matmul_call(x[c * CH:(c + 1) * CH], w)
        wt, it = _topk_kernel(lg)
        ws.append(wt.transpose(0, 2, 1).reshape(CH, TOPK))
        inds.append(it.transpose(0, 2, 1).reshape(CH, TOPK))
    return jnp.concatenate(ws, axis=0), jnp.concatenate(inds, axis=0)


# speedup vs baseline: 2.4706x; 2.4706x over previous
import functools

import jax
import jax.numpy as jnp
from jax import lax
from jax.experimental import pallas as pl
from jax.experimental.pallas import tpu as pltpu
from jax.experimental.pallas import tpu_sc as plsc

HID = 4096
NE = 64
TOPK = 8
NTOK = 8192
BLK = 512
NC, NS, L = 2, 16, 16
NW = NC * NS
TPW = NTOK // NW

_SORT8 = [(0, 1), (2, 3), (4, 5), (6, 7),
          (0, 2), (1, 3), (4, 6), (5, 7),
          (1, 2), (5, 6),
          (0, 4), (1, 5), (2, 6), (3, 7),
          (2, 4), (3, 5),
          (1, 2), (3, 4), (5, 6)]
_BITONIC8 = [(0, 4), (1, 5), (2, 6), (3, 7),
             (0, 2), (1, 3), (4, 6), (5, 7),
             (0, 1), (2, 3), (4, 5), (6, 7)]


def _matmul_kernel(x_ref, w_ref, out_ref):
    logits = lax.dot_general(
        w_ref[...], x_ref[...], (((1,), (1,)), ((), ())),
        preferred_element_type=jnp.float32)
    for c in range(BLK // TPW):
        out_ref[c] = logits[:, c * TPW:(c + 1) * TPW]


def _ce(v, ix, i, j):
    c = v[i] >= v[j]
    hi = jnp.maximum(v[i], v[j])
    lo = jnp.minimum(v[i], v[j])
    ihi = jnp.where(c, ix[i], ix[j])
    ilo = jnp.where(c, ix[j], ix[i])
    v[i], v[j] = hi, lo
    ix[i], ix[j] = ihi, ilo


def _merge_top8(av, ai, bv, bi):
    cv, ci = [], []
    for i in range(TOPK):
        c = av[i] >= bv[TOPK - 1 - i]
        cv.append(jnp.maximum(av[i], bv[TOPK - 1 - i]))
        ci.append(jnp.where(c, ai[i], bi[TOPK - 1 - i]))
    for (i, j) in _BITONIC8:
        _ce(cv, ci, i, j)
    return cv, ci


_sc_mesh = plsc.VectorSubcoreMesh(core_axis_name="c", subcore_axis_name="s")


@functools.partial(
    pl.kernel,
    mesh=_sc_mesh,
    out_type=[
        jax.ShapeDtypeStruct((NW, TOPK, TPW), jnp.float32),
        jax.ShapeDtypeStruct((NW, TOPK, TPW), jnp.int32),
    ],
    scratch_types=[
        pltpu.VMEM((NE, TPW), jnp.float32),
        pltpu.VMEM((TOPK, TPW), jnp.float32),
        pltpu.VMEM((TOPK, TPW), jnp.int32),
    ],
)
def _topk_kernel(lg_hbm, wout_hbm, iout_hbm, lg_v, wv, iv):
    wid = lax.axis_index("s") * NC + lax.axis_index("c")
    pltpu.sync_copy(lg_hbm.at[wid], lg_v)

    def group(g, carry):
        off = g * L
        chunks = []
        for c in range(NE // TOPK):
            v = [lg_v[c * TOPK + e, pl.ds(off, L)] for e in range(TOPK)]
            ix = [jnp.full((L,), c * TOPK + e, jnp.int32)
                  for e in range(TOPK)]
            for (i, j) in _SORT8:
                _ce(v, ix, i, j)
            chunks.append((v, ix))
        while len(chunks) > 1:
            nxt = []
            for k in range(0, len(chunks), 2):
                (av, ai), (bv, bi) = chunks[k], chunks[k + 1]
                nxt.append(_merge_top8(av, ai, bv, bi))
            chunks = nxt
        vals, idxs = chunks[0]
        es = [jnp.exp(v - vals[0]) for v in vals]
        s = es[0]
        for j in range(1, TOPK):
            s = s + es[j]
        for j in range(TOPK):
            wv[j, pl.ds(off, L)] = es[j] / s
            iv[j, pl.ds(off, L)] = idxs[j]
        return carry

    lax.fori_loop(0, TPW // L, group, 0)
    pltpu.sync_copy(wv, wout_hbm.at[wid])
    pltpu.sync_copy(iv, iout_hbm.at[wid])


@jax.jit
def kernel(hidden_states, weight):
    b, s, h = hidden_states.shape
    n = b * s
    x = hidden_states.reshape(n, h).astype(jnp.float32)
    lg = pl.pallas_call(
        _matmul_kernel,
        grid=(n // BLK,),
        in_specs=[
            pl.BlockSpec((BLK, HID), lambda t: (t, 0)),
            pl.BlockSpec((NE, HID), lambda t: (0, 0)),
        ],
        out_specs=pl.BlockSpec((BLK // TPW, NE, TPW), lambda t: (t, 0, 0)),
        out_shape=jax.ShapeDtypeStruct((n // TPW, NE, TPW), jnp.float32),
        compiler_params=pltpu.CompilerParams(
            dimension_semantics=("arbitrary",)),
    )(x, weight.astype(jnp.float32))
    wt, it = _topk_kernel(lg)
    wout = wt.transpose(0, 2, 1).reshape(n, TOPK)
    iout = it.transpose(0, 2, 1).reshape(n, TOPK)
    return wout, iout


# speedup vs baseline: 2.4710x; 1.0001x over previous
import functools

import jax
import jax.numpy as jnp
from jax import lax
from jax.experimental import pallas as pl
from jax.experimental.pallas import tpu as pltpu
from jax.experimental.pallas import tpu_sc as plsc

HID = 4096
NE = 64
TOPK = 8
NTOK = 8192
BLK = 512
NC, NS, L = 2, 16, 16
NW = NC * NS
TPW = NTOK // NW

_SORT8 = [(0, 1), (2, 3), (4, 5), (6, 7),
          (0, 2), (1, 3), (4, 6), (5, 7),
          (1, 2), (5, 6),
          (0, 4), (1, 5), (2, 6), (3, 7),
          (2, 4), (3, 5),
          (1, 2), (3, 4), (5, 6)]
_BITONIC8 = [(0, 4), (1, 5), (2, 6), (3, 7),
             (0, 2), (1, 3), (4, 6), (5, 7),
             (0, 1), (2, 3), (4, 5), (6, 7)]


KS = 2
KC = HID // KS


def _matmul_kernel(*refs):
    x_refs, w_ref, out_ref = refs[:KS], refs[KS], refs[KS + 1]
    logits = None
    for k in range(KS):
        part = lax.dot_general(
            w_ref[:, k * KC:(k + 1) * KC], x_refs[k][...],
            (((1,), (1,)), ((), ())),
            preferred_element_type=jnp.float32)
        logits = part if logits is None else logits + part
    for c in range(BLK // TPW):
        out_ref[c] = logits[:, c * TPW:(c + 1) * TPW]


def _ce(v, ix, i, j):
    c = v[i] >= v[j]
    hi = jnp.maximum(v[i], v[j])
    lo = jnp.minimum(v[i], v[j])
    ihi = jnp.where(c, ix[i], ix[j])
    ilo = jnp.where(c, ix[j], ix[i])
    v[i], v[j] = hi, lo
    ix[i], ix[j] = ihi, ilo


def _merge_top8(av, ai, bv, bi):
    cv, ci = [], []
    for i in range(TOPK):
        c = av[i] >= bv[TOPK - 1 - i]
        cv.append(jnp.maximum(av[i], bv[TOPK - 1 - i]))
        ci.append(jnp.where(c, ai[i], bi[TOPK - 1 - i]))
    for (i, j) in _BITONIC8:
        _ce(cv, ci, i, j)
    return cv, ci


_sc_mesh = plsc.VectorSubcoreMesh(core_axis_name="c", subcore_axis_name="s")


@functools.partial(
    pl.kernel,
    mesh=_sc_mesh,
    out_type=[
        jax.ShapeDtypeStruct((NW, TOPK, TPW), jnp.float32),
        jax.ShapeDtypeStruct((NW, TOPK, TPW), jnp.int32),
    ],
    scratch_types=[
        pltpu.VMEM((NE, TPW), jnp.float32),
        pltpu.VMEM((TOPK, TPW), jnp.float32),
        pltpu.VMEM((TOPK, TPW), jnp.int32),
    ],
)
def _topk_kernel(lg_hbm, wout_hbm, iout_hbm, lg_v, wv, iv):
    wid = lax.axis_index("s") * NC + lax.axis_index("c")
    pltpu.sync_copy(lg_hbm.at[wid], lg_v)

    def group(g, carry):
        off = g * L
        chunks = []
        for c in range(NE // TOPK):
            v = [lg_v[c * TOPK + e, pl.ds(off, L)] for e in range(TOPK)]
            ix = [jnp.full((L,), c * TOPK + e, jnp.int32)
                  for e in range(TOPK)]
            for (i, j) in _SORT8:
                _ce(v, ix, i, j)
            chunks.append((v, ix))
        while len(chunks) > 1:
            nxt = []
            for k in range(0, len(chunks), 2):
                (av, ai), (bv, bi) = chunks[k], chunks[k + 1]
                nxt.append(_merge_top8(av, ai, bv, bi))
            chunks = nxt
        vals, idxs = chunks[0]
        es = [jnp.exp(v - vals[0]) for v in vals]
        s = es[0]
        for j in range(1, TOPK):
            s = s + es[j]
        for j in range(TOPK):
            wv[j, pl.ds(off, L)] = es[j] / s
            iv[j, pl.ds(off, L)] = idxs[j]
        return carry

    lax.fori_loop(0, TPW // L, group, 0)
    pltpu.sync_copy(wv, wout_hbm.at[wid])
    pltpu.sync_copy(iv, iout_hbm.at[wid])


@jax.jit
def kernel(hidden_states, weight):
    b, s, h = hidden_states.shape
    n = b * s
    x = hidden_states.reshape(n, h).astype(jnp.float32)
    lg = pl.pallas_call(
        _matmul_kernel,
        grid=(n // BLK,),
        in_specs=[
            *[pl.BlockSpec((BLK, KC), functools.partial(
                lambda k, t: (t, k), k)) for k in range(KS)],
            pl.BlockSpec((NE, HID), lambda t: (0, 0)),
        ],
        out_specs=pl.BlockSpec((BLK // TPW, NE, TPW), lambda t: (t, 0, 0)),
        out_shape=jax.ShapeDtypeStruct((n // TPW, NE, TPW), jnp.float32),
        compiler_params=pltpu.CompilerParams(
            dimension_semantics=("arbitrary",)),
    )(*([x] * KS), weight.astype(jnp.float32))
    wt, it = _topk_kernel(lg)
    wout = wt.transpose(0, 2, 1).reshape(n, TOPK)
    iout = it.transpose(0, 2, 1).reshape(n, TOPK)
    return wout, iout


# speedup vs baseline: 2.4832x; 1.0050x over previous
import functools

import jax
import jax.numpy as jnp
from jax import lax
from jax.experimental import pallas as pl
from jax.experimental.pallas import tpu as pltpu
from jax.experimental.pallas import tpu_sc as plsc

HID = 4096
NE = 64
TOPK = 8
NTOK = 8192
BLK = 512
NC, NS, L = 2, 16, 16
NW = NC * NS
TPW = NTOK // NW

_SORT8 = [(0, 1), (2, 3), (4, 5), (6, 7),
          (0, 2), (1, 3), (4, 6), (5, 7),
          (1, 2), (5, 6),
          (0, 4), (1, 5), (2, 6), (3, 7),
          (2, 4), (3, 5),
          (1, 2), (3, 4), (5, 6)]
_BITONIC8 = [(0, 4), (1, 5), (2, 6), (3, 7),
             (0, 2), (1, 3), (4, 6), (5, 7),
             (0, 1), (2, 3), (4, 5), (6, 7)]


def _matmul_kernel(x_ref, w_ref, out_ref):
    logits = lax.dot_general(
        w_ref[...], x_ref[...], (((1,), (1,)), ((), ())),
        preferred_element_type=jnp.float32)
    for c in range(BLK // TPW):
        out_ref[c] = logits[:, c * TPW:(c + 1) * TPW]


def _ce(v, ix, i, j):
    c = v[i] >= v[j]
    hi = jnp.maximum(v[i], v[j])
    lo = jnp.minimum(v[i], v[j])
    ihi = jnp.where(c, ix[i], ix[j])
    ilo = jnp.where(c, ix[j], ix[i])
    v[i], v[j] = hi, lo
    ix[i], ix[j] = ihi, ilo


def _merge_top8(av, ai, bv, bi):
    cv, ci = [], []
    for i in range(TOPK):
        c = av[i] >= bv[TOPK - 1 - i]
        cv.append(jnp.maximum(av[i], bv[TOPK - 1 - i]))
        ci.append(jnp.where(c, ai[i], bi[TOPK - 1 - i]))
    for (i, j) in _BITONIC8:
        _ce(cv, ci, i, j)
    return cv, ci


_sc_mesh = plsc.VectorSubcoreMesh(core_axis_name="c", subcore_axis_name="s")


@functools.partial(
    pl.kernel,
    mesh=_sc_mesh,
    out_type=[
        jax.ShapeDtypeStruct((NW, TOPK, TPW), jnp.float32),
        jax.ShapeDtypeStruct((NW, TOPK, TPW), jnp.int32),
    ],
    scratch_types=[
        pltpu.VMEM((NE, TPW), jnp.float32),
        pltpu.VMEM((TOPK, TPW), jnp.float32),
        pltpu.VMEM((TOPK, TPW), jnp.int32),
    ],
)
def _topk_kernel(lg_hbm, wout_hbm, iout_hbm, lg_v, wv, iv):
    wid = lax.axis_index("s") * NC + lax.axis_index("c")
    pltpu.sync_copy(lg_hbm.at[wid], lg_v)

    def group(g, carry):
        off = g * L
        chunks = []
        for c in range(NE // TOPK):
            v = [lg_v[c * TOPK + e, pl.ds(off, L)] for e in range(TOPK)]
            ix = [jnp.full((L,), c * TOPK + e, jnp.int32)
                  for e in range(TOPK)]
            for (i, j) in _SORT8:
                _ce(v, ix, i, j)
            chunks.append((v, ix))
        while len(chunks) > 1:
            nxt = []
            for k in range(0, len(chunks), 2):
                (av, ai), (bv, bi) = chunks[k], chunks[k + 1]
                nxt.append(_merge_top8(av, ai, bv, bi))
            chunks = nxt
        vals, idxs = chunks[0]
        es = [jnp.exp(v - vals[0]) for v in vals]
        s = es[0]
        for j in range(1, TOPK):
            s = s + es[j]
        for j in range(TOPK):
            wv[j, pl.ds(off, L)] = es[j] / s
            iv[j, pl.ds(off, L)] = idxs[j]
        return carry

    lax.fori_loop(0, TPW // L, group, 0)
    pltpu.sync_copy(wv, wout_hbm.at[wid])
    pltpu.sync_copy(iv, iout_hbm.at[wid])


@jax.jit
def kernel(hidden_states, weight):
    b, s, h = hidden_states.shape
    n = b * s
    x = hidden_states.reshape(n, h).astype(jnp.float32)
    lg = pl.pallas_call(
        _matmul_kernel,
        grid=(n // BLK,),
        in_specs=[
            pl.BlockSpec((BLK, HID), lambda t: (t, 0)),
            pl.BlockSpec((NE, HID), lambda t: (0, 0)),
        ],
        out_specs=pl.BlockSpec((BLK // TPW, NE, TPW), lambda t: (t, 0, 0)),
        out_shape=jax.ShapeDtypeStruct((n // TPW, NE, TPW), jnp.float32),
        compiler_params=pltpu.CompilerParams(
            dimension_semantics=("arbitrary",)),
    )(x, weight.astype(jnp.float32))
    wt, it = _topk_kernel(lg)
    wout = wt.transpose(0, 2, 1).reshape(n, TOPK)
    iout = it.transpose(0, 2, 1).reshape(n, TOPK)
    return wout, iout
